# baseline (speedup 1.0000x reference)
import jax
import jax.numpy as jnp
from jax import lax
from jax.experimental import pallas as pl
from jax.experimental.pallas import tpu as pltpu
from jax.experimental.pallas import tpu_sc as plsc

_B, _C, _W, _H = 8, 96, 224, 224
_N = _W * _H
_TOT = _B * _C * _N
_NB = 4096
_LO, _HI = -5.5, 5.5
_WIDTH = (_HI - _LO) / _NB
_INVW = 1.0 / _WIDTH
_Q = 25216
_CH = 6272
_NCH = _N // _CH
_VPC = _CH // 16
_NW = 32
_CPW = _C // _NW


def _efd_body(x_hbm, lam_hbm, perm_hbm, out_hbm, hist, rqf, rlut,
              i0, i1, o0, o1, lamb, permb, si0, si1, so0, so1):
    wid = lax.axis_index("s") * 2 + lax.axis_index("c")
    ones = jnp.ones((16,), jnp.int32)
    zeros = jnp.zeros((16,), jnp.int32)
    lanes = lax.iota(jnp.int32, 16)
    full15 = jnp.full((16, 1), 15, jnp.int32)
    nextlane = jnp.minimum(lax.iota(jnp.int32, 16) + 1, 15).reshape(16, 1)
    _gd = lax.GatherDimensionNumbers(
        offset_dims=(), collapsed_slice_dims=(0,), start_index_map=(0,))
    _gd16 = _gd

    def bcast_last(v):
        return lax.gather(v, full15, _gd, (1,),
                          mode=lax.GatherScatterMode.PROMISE_IN_BOUNDS)

    def start_in(off, buf, sem):
        return pltpu.async_copy(x_hbm.at[pl.ds(off, _CH)], buf, sem)

    def wait_in(off, buf, sem):
        pltpu.make_async_copy(x_hbm.at[pl.ds(off, _CH)], buf, sem).wait()

    def start_out(off, buf, sem):
        return pltpu.async_copy(buf, out_hbm.at[pl.ds(off, _CH)], sem)

    def wait_out(off, buf, sem):
        pltpu.make_async_copy(buf, out_hbm.at[pl.ds(off, _CH)], sem).wait()

    @plsc.parallel_loop(0, _Q // 16, unroll=8)
    def _(i):
        rlut[pl.ds(i * 16, 16)] = zeros

    def channel_body(t, _):
        chan = t * _NW + wid

        @plsc.parallel_loop(0, (_B * _NB) // 16, unroll=8)
        def _(i):
            hist[pl.ds(i * 16, 16)] = zeros

        def hist_chunk(buf, hb):
            @plsc.parallel_loop(0, _VPC, unroll=8)
            def _(i):
                v = buf[pl.ds(i * 16, 16)]
                tt = jnp.clip((v - _LO) * _INVW, 0.0, float(_NB - 1))
                k = tt.astype(jnp.int32) + hb
                plsc.addupdate_scatter(hist, [k], ones)

        def p1_row(b, _):
            base = (b * _C + chan) * _N
            hb = b * _NB
            start_in(base, i0, si0)
            start_in(base + _CH, i1, si1)

            def p1_pair(p, _):
                off0 = base + (2 * p) * _CH
                wait_in(off0, i0, si0)
                hist_chunk(i0, hb)

                @pl.when(2 * p + 2 < _NCH)
                def _():
                    start_in(off0 + 2 * _CH, i0, si0)

                wait_in(off0 + _CH, i1, si1)
                hist_chunk(i1, hb)

                @pl.when(2 * p + 3 < _NCH)
                def _():
                    start_in(off0 + 3 * _CH, i1, si1)

                return 0

            lax.fori_loop(0, _NCH // 2, p1_pair, 0)
            return 0

        lax.fori_loop(0, _B, p1_row, 0)

        def p2_row(b, _):
            pltpu.sync_copy(perm_hbm.at[b], permb)
            pb = jnp.max(permb[...])
            tag = (t * _B + b) * _NB

            pbase = pb * _NB

            def p2_scatter(i, carry):
                h = hist[pl.ds(pbase + i * 16, 16)]
                ci = plsc.cumsum(h) + carry
                kv = (i * 16 + tag) + lanes
                pos = (ci - h) >> 1
                pnext = lax.gather(
                    pos, nextlane, _gd16, (1,),
                    mode=lax.GatherScatterMode.PROMISE_IN_BOUNDS)
                keep = (pos != pnext) | (lanes == 15)
                plsc.store_scatter(rlut, [pos], kv, mask=(h > 0) & keep)
                return bcast_last(ci)

            lax.fori_loop(0, _NB // 16, p2_scatter, zeros, unroll=8)

            def p2_cummax(i, carry):
                v = rlut[pl.ds(i * 16, 16)]
                cm = jnp.maximum(plsc.cummax(v), carry)
                rlut[pl.ds(i * 16, 16)] = cm
                return bcast_last(cm)

            lax.fori_loop(0, _Q // 16, p2_cummax, zeros + tag, unroll=8)

            bbase = b * _NB

            def p2_rmid(i, carry):
                h = hist[pl.ds(bbase + i * 16, 16)]
                ci = plsc.cumsum(h) + carry
                rmid = ci - h + (h >> 1)
                rqf[pl.ds(bbase + i * 16, 16)] = plsc.bitcast(
                    rmid, jnp.float32)
                return bcast_last(ci)

            lax.fori_loop(0, _NB // 16, p2_rmid, zeros, unroll=8)

            @plsc.parallel_loop(0, _NB // 16, unroll=8)
            def _(i):
                rmid = plsc.bitcast(
                    rqf[pl.ds(bbase + i * 16, 16)], jnp.int32)
                j = plsc.load_gather(rlut, [rmid >> 1]) - tag
                rqf[pl.ds(bbase + i * 16, 16)] = (
                    _LO + _WIDTH * (j.astype(jnp.float32) + 0.5))

            return 0

        lax.fori_loop(0, _B, p2_row, 0)

        def mix_chunk(ib, ob, bbase, oml):
            @plsc.parallel_loop(0, _VPC, unroll=8)
            def _(i):
                v = ib[pl.ds(i * 16, 16)]
                tt = jnp.clip((v - _LO) * _INVW, 0.0, float(_NB - 1))
                k = tt.astype(jnp.int32) + bbase
                m = plsc.load_gather(rqf, [k])
                ob[pl.ds(i * 16, 16)] = v + (m * oml - v * oml)

        def p3_row(b, _):
            base = (b * _C + chan) * _N
            bbase = b * _NB
            pltpu.sync_copy(lam_hbm.at[b], lamb)
            oml = 1.0 - lamb[...]
            start_in(base, i0, si0)
            start_in(base + _CH, i1, si1)

            def p3_pair(p, _):
                off0 = base + (2 * p) * _CH
                wait_in(off0, i0, si0)

                @pl.when(p > 0)
                def _():
                    wait_out(off0 - 2 * _CH, o0, so0)

                mix_chunk(i0, o0, bbase, oml)
                start_out(off0, o0, so0)

                @pl.when(2 * p + 2 < _NCH)
                def _():
                    start_in(off0 + 2 * _CH, i0, si0)

                wait_in(off0 + _CH, i1, si1)

                @pl.when(p > 0)
                def _():
                    wait_out(off0 - _CH, o1, so1)

                mix_chunk(i1, o1, bbase, oml)
                start_out(off0 + _CH, o1, so1)

                @pl.when(2 * p + 3 < _NCH)
                def _():
                    start_in(off0 + 3 * _CH, i1, si1)

                return 0

            lax.fori_loop(0, _NCH // 2, p3_pair, 0)
            wait_out(base + (_NCH - 2) * _CH, o0, so0)
            wait_out(base + (_NCH - 1) * _CH, o1, so1)
            return 0

        lax.fori_loop(0, _B, p3_row, 0)
        return 0

    lax.fori_loop(0, _CPW, channel_body, 0)


_efd_call = pl.kernel(
    _efd_body,
    out_type=jax.ShapeDtypeStruct((_TOT,), jnp.float32),
    mesh=plsc.VectorSubcoreMesh(core_axis_name="c", subcore_axis_name="s"),
    compiler_params=pltpu.CompilerParams(needs_layout_passes=False),
    scratch_types=[
        pltpu.VMEM((_B * _NB,), jnp.int32),
        pltpu.VMEM((_B * _NB,), jnp.float32),
        pltpu.VMEM((_Q,), jnp.int32),
        pltpu.VMEM((_CH,), jnp.float32),
        pltpu.VMEM((_CH,), jnp.float32),
        pltpu.VMEM((_CH,), jnp.float32),
        pltpu.VMEM((_CH,), jnp.float32),
        pltpu.VMEM((16,), jnp.float32),
        pltpu.VMEM((16,), jnp.int32),
        pltpu.SemaphoreType.DMA,
        pltpu.SemaphoreType.DMA,
        pltpu.SemaphoreType.DMA,
        pltpu.SemaphoreType.DMA,
    ],
)


def kernel(x):
    B, C, W, H = x.shape
    k_beta, k_perm = jax.random.split(jax.random.key(42))
    lmda = jax.random.beta(k_beta, 0.1, 0.1, (B, 1, 1)).astype(x.dtype)
    perm = jax.random.permutation(k_perm, B)
    lam16 = jnp.broadcast_to(lmda.reshape(B, 1), (B, 16)).astype(jnp.float32)
    perm16 = jnp.broadcast_to(
        perm.reshape(B, 1).astype(jnp.int32), (B, 16))
    out = _efd_call(x.reshape(-1), lam16, perm16)
    return out.reshape(B, C, W, H)


# speedup vs baseline: 1.0457x; 1.0457x over previous
import jax
import jax.numpy as jnp
from jax import lax
from jax.experimental import pallas as pl
from jax.experimental.pallas import tpu as pltpu
from jax.experimental.pallas import tpu_sc as plsc

_B, _C, _W, _H = 8, 96, 224, 224
_N = _W * _H
_TOT = _B * _C * _N
_NB = 4096
_LO, _HI = -5.5, 5.5
_WIDTH = (_HI - _LO) / _NB
_INVW = 1.0 / _WIDTH
_Q = 12672
_CH = 12544
_NCH = _N // _CH
_VPC = _CH // 16
_NW = 32
_CPW = _C // _NW


def _efd_body(x_hbm, lam_hbm, perm_hbm, out_hbm, hist, rqf, rlut,
              i0, i1, o0, o1, lamb, permb, si0, si1, so0, so1):
    wid = lax.axis_index("s") * 2 + lax.axis_index("c")
    ones = jnp.ones((16,), jnp.int32)
    zeros = jnp.zeros((16,), jnp.int32)
    lanes = lax.iota(jnp.int32, 16)
    full15 = jnp.full((16, 1), 15, jnp.int32)
    nextlane = jnp.minimum(lax.iota(jnp.int32, 16) + 1, 15).reshape(16, 1)
    _gd = lax.GatherDimensionNumbers(
        offset_dims=(), collapsed_slice_dims=(0,), start_index_map=(0,))
    _gd16 = _gd

    def bcast_last(v):
        return lax.gather(v, full15, _gd, (1,),
                          mode=lax.GatherScatterMode.PROMISE_IN_BOUNDS)

    def start_in(off, buf, sem):
        return pltpu.async_copy(x_hbm.at[pl.ds(off, _CH)], buf, sem)

    def wait_in(off, buf, sem):
        pltpu.make_async_copy(x_hbm.at[pl.ds(off, _CH)], buf, sem).wait()

    def start_out(off, buf, sem):
        return pltpu.async_copy(buf, out_hbm.at[pl.ds(off, _CH)], sem)

    def wait_out(off, buf, sem):
        pltpu.make_async_copy(buf, out_hbm.at[pl.ds(off, _CH)], sem).wait()

    @plsc.parallel_loop(0, _Q // 16, unroll=8)
    def _(i):
        rlut[pl.ds(i * 16, 16)] = zeros

    def channel_body(t, _):
        chan = t * _NW + wid

        @plsc.parallel_loop(0, (_B * _NB) // 16, unroll=8)
        def _(i):
            hist[pl.ds(i * 16, 16)] = zeros

        def hist_chunk(buf, hb):
            @plsc.parallel_loop(0, _VPC, unroll=8)
            def _(i):
                v = buf[pl.ds(i * 16, 16)]
                tt = jnp.clip((v - _LO) * _INVW, 0.0, float(_NB - 1))
                k = tt.astype(jnp.int32) + hb
                plsc.addupdate_scatter(hist, [k], ones)

        def p1_row(b, _):
            base = (b * _C + chan) * _N
            hb = b * _NB
            start_in(base, i0, si0)
            start_in(base + _CH, i1, si1)

            def p1_pair(p, _):
                off0 = base + (2 * p) * _CH
                wait_in(off0, i0, si0)
                hist_chunk(i0, hb)

                @pl.when(2 * p + 2 < _NCH)
                def _():
                    start_in(off0 + 2 * _CH, i0, si0)

                wait_in(off0 + _CH, i1, si1)
                hist_chunk(i1, hb)

                @pl.when(2 * p + 3 < _NCH)
                def _():
                    start_in(off0 + 3 * _CH, i1, si1)

                return 0

            lax.fori_loop(0, _NCH // 2, p1_pair, 0)
            return 0

        lax.fori_loop(0, _B, p1_row, 0)

        def p2_row(b, _):
            pltpu.sync_copy(perm_hbm.at[b], permb)
            pb = jnp.max(permb[...])
            tag = (t * _B + b) * _NB

            pbase = pb * _NB

            def p2_scatter(i, carry):
                h = hist[pl.ds(pbase + i * 16, 16)]
                ci = plsc.cumsum(h) + carry
                kv = (i * 16 + tag) + lanes
                pos = (ci - h) >> 2
                pnext = lax.gather(
                    pos, nextlane, _gd16, (1,),
                    mode=lax.GatherScatterMode.PROMISE_IN_BOUNDS)
                keep = (pos != pnext) | (lanes == 15)
                plsc.store_scatter(rlut, [pos], kv, mask=(h > 0) & keep)
                return bcast_last(ci)

            lax.fori_loop(0, _NB // 16, p2_scatter, zeros, unroll=8)

            def p2_cummax(i, carry):
                v = rlut[pl.ds(i * 16, 16)]
                cm = jnp.maximum(plsc.cummax(v), carry)
                rlut[pl.ds(i * 16, 16)] = cm
                return bcast_last(cm)

            lax.fori_loop(0, _Q // 16, p2_cummax, zeros + tag, unroll=8)

            bbase = b * _NB

            def p2_rmid(i, carry):
                h = hist[pl.ds(bbase + i * 16, 16)]
                ci = plsc.cumsum(h) + carry
                rmid = ci - h + (h >> 1)
                rqf[pl.ds(bbase + i * 16, 16)] = plsc.bitcast(
                    rmid, jnp.float32)
                return bcast_last(ci)

            lax.fori_loop(0, _NB // 16, p2_rmid, zeros, unroll=8)

            @plsc.parallel_loop(0, _NB // 16, unroll=8)
            def _(i):
                rmid = plsc.bitcast(
                    rqf[pl.ds(bbase + i * 16, 16)], jnp.int32)
                j = plsc.load_gather(rlut, [rmid >> 2]) - tag
                rqf[pl.ds(bbase + i * 16, 16)] = (
                    _LO + _WIDTH * (j.astype(jnp.float32) + 0.5))

            return 0

        lax.fori_loop(0, _B, p2_row, 0)

        def mix_chunk(ib, ob, bbase, oml):
            @plsc.parallel_loop(0, _VPC, unroll=8)
            def _(i):
                v = ib[pl.ds(i * 16, 16)]
                tt = jnp.clip((v - _LO) * _INVW, 0.0, float(_NB - 1))
                k = tt.astype(jnp.int32) + bbase
                m = plsc.load_gather(rqf, [k])
                ob[pl.ds(i * 16, 16)] = v + (m * oml - v * oml)

        def p3_row(b, _):
            base = (b * _C + chan) * _N
            bbase = b * _NB
            pltpu.sync_copy(lam_hbm.at[b], lamb)
            oml = 1.0 - lamb[...]
            start_in(base, i0, si0)
            start_in(base + _CH, i1, si1)

            def p3_pair(p, _):
                off0 = base + (2 * p) * _CH
                wait_in(off0, i0, si0)

                @pl.when(p > 0)
                def _():
                    wait_out(off0 - 2 * _CH, o0, so0)

                mix_chunk(i0, o0, bbase, oml)
                start_out(off0, o0, so0)

                @pl.when(2 * p + 2 < _NCH)
                def _():
                    start_in(off0 + 2 * _CH, i0, si0)

                wait_in(off0 + _CH, i1, si1)

                @pl.when(p > 0)
                def _():
                    wait_out(off0 - _CH, o1, so1)

                mix_chunk(i1, o1, bbase, oml)
                start_out(off0 + _CH, o1, so1)

                @pl.when(2 * p + 3 < _NCH)
                def _():
                    start_in(off0 + 3 * _CH, i1, si1)

                return 0

            lax.fori_loop(0, _NCH // 2, p3_pair, 0)
            wait_out(base + (_NCH - 2) * _CH, o0, so0)
            wait_out(base + (_NCH - 1) * _CH, o1, so1)
            return 0

        lax.fori_loop(0, _B, p3_row, 0)
        return 0

    lax.fori_loop(0, _CPW, channel_body, 0)


_efd_call = pl.kernel(
    _efd_body,
    out_type=jax.ShapeDtypeStruct((_TOT,), jnp.float32),
    mesh=plsc.VectorSubcoreMesh(core_axis_name="c", subcore_axis_name="s"),
    compiler_params=pltpu.CompilerParams(needs_layout_passes=False),
    scratch_types=[
        pltpu.VMEM((_B * _NB,), jnp.int32),
        pltpu.VMEM((_B * _NB,), jnp.float32),
        pltpu.VMEM((_Q,), jnp.int32),
        pltpu.VMEM((_CH,), jnp.float32),
        pltpu.VMEM((_CH,), jnp.float32),
        pltpu.VMEM((_CH,), jnp.float32),
        pltpu.VMEM((_CH,), jnp.float32),
        pltpu.VMEM((16,), jnp.float32),
        pltpu.VMEM((16,), jnp.int32),
        pltpu.SemaphoreType.DMA,
        pltpu.SemaphoreType.DMA,
        pltpu.SemaphoreType.DMA,
        pltpu.SemaphoreType.DMA,
    ],
)


def kernel(x):
    B, C, W, H = x.shape
    k_beta, k_perm = jax.random.split(jax.random.key(42))
    lmda = jax.random.beta(k_beta, 0.1, 0.1, (B, 1, 1)).astype(x.dtype)
    perm = jax.random.permutation(k_perm, B)
    lam16 = jnp.broadcast_to(lmda.reshape(B, 1), (B, 16)).astype(jnp.float32)
    perm16 = jnp.broadcast_to(
        perm.reshape(B, 1).astype(jnp.int32), (B, 16))
    out = _efd_call(x.reshape(-1), lam16, perm16)
    return out.reshape(B, C, W, H)


# speedup vs baseline: 1.2990x; 1.2422x over previous
import jax
import jax.numpy as jnp
from jax import lax
from jax.experimental import pallas as pl
from jax.experimental.pallas import tpu as pltpu
from jax.experimental.pallas import tpu_sc as plsc

_B, _C, _W, _H = 8, 96, 224, 224
_N = _W * _H
_NB = 4096
_LO, _HI = -5.5, 5.5
_WIDTH = (_HI - _LO) / _NB
_INVW = 1.0 / _WIDTH
_Q = 12672
_CHR = 16
_NCH = _W // _CHR
_CVEC = _H // 16
_NW = 32
_CPW = _C // _NW


def _efd_body(x_hbm, lam_hbm, perm_hbm, out_hbm, hist, rqf, rlut,
              i0, i1, o0, o1, lamb, permb, si0, si1, so0, so1):
    wid = lax.axis_index("s") * 2 + lax.axis_index("c")
    ones = jnp.ones((16,), jnp.int32)
    zeros = jnp.zeros((16,), jnp.int32)
    lanes = lax.iota(jnp.int32, 16)
    full15 = jnp.full((16, 1), 15, jnp.int32)
    nextlane = jnp.minimum(lax.iota(jnp.int32, 16) + 1, 15).reshape(16, 1)
    _gd = lax.GatherDimensionNumbers(
        offset_dims=(), collapsed_slice_dims=(0,), start_index_map=(0,))

    def bcast_last(v):
        return lax.gather(v, full15, _gd, (1,),
                          mode=lax.GatherScatterMode.PROMISE_IN_BOUNDS)

    def start_in(b, c, j, buf, sem):
        pltpu.async_copy(x_hbm.at[b, c, pl.ds(j * _CHR, _CHR)], buf, sem)

    def wait_in(b, c, j, buf, sem):
        pltpu.make_async_copy(
            x_hbm.at[b, c, pl.ds(j * _CHR, _CHR)], buf, sem).wait()

    def start_out(b, c, j, buf, sem):
        pltpu.async_copy(buf, out_hbm.at[b, c, pl.ds(j * _CHR, _CHR)], sem)

    def wait_out(b, c, j, buf, sem):
        pltpu.make_async_copy(
            buf, out_hbm.at[b, c, pl.ds(j * _CHR, _CHR)], sem).wait()

    @plsc.parallel_loop(0, _Q // 16, unroll=8)
    def _(i):
        rlut[pl.ds(i * 16, 16)] = zeros

    def channel_body(t, _):
        chan = t * _NW + wid

        @plsc.parallel_loop(0, (_B * _NB) // 16, unroll=8)
        def _(i):
            hist[pl.ds(i * 16, 16)] = zeros

        def hist_chunk(buf, hb):
            @plsc.parallel_loop(0, _CHR)
            def _(r):
                for cc in range(_CVEC):
                    v = buf[r, pl.ds(cc * 16, 16)]
                    tt = jnp.clip((v - _LO) * _INVW, 0.0, float(_NB - 1))
                    k = tt.astype(jnp.int32) + hb
                    plsc.addupdate_scatter(hist, [k], ones)

        def p1_row(b, _):
            hb = b * _NB
            start_in(b, chan, 0, i0, si0)
            start_in(b, chan, 1, i1, si1)

            def p1_pair(p, _):
                wait_in(b, chan, 2 * p, i0, si0)
                hist_chunk(i0, hb)

                @pl.when(2 * p + 2 < _NCH)
                def _():
                    start_in(b, chan, 2 * p + 2, i0, si0)

                wait_in(b, chan, 2 * p + 1, i1, si1)
                hist_chunk(i1, hb)

                @pl.when(2 * p + 3 < _NCH)
                def _():
                    start_in(b, chan, 2 * p + 3, i1, si1)

                return 0

            lax.fori_loop(0, _NCH // 2, p1_pair, 0)
            return 0

        lax.fori_loop(0, _B, p1_row, 0)

        def p2_row(b, _):
            pltpu.sync_copy(perm_hbm.at[b], permb)
            pb = jnp.max(permb[pl.ds(0, 16)])
            tag = (t * _B + b) * _NB

            pbase = pb * _NB

            def p2_scatter(i, carry):
                h = hist[pl.ds(pbase + i * 16, 16)]
                ci = plsc.cumsum(h) + carry
                kv = (i * 16 + tag) + lanes
                pos = (ci - h) >> 2
                pnext = lax.gather(
                    pos, nextlane, _gd, (1,),
                    mode=lax.GatherScatterMode.PROMISE_IN_BOUNDS)
                keep = (pos != pnext) | (lanes == 15)
                plsc.store_scatter(rlut, [pos], kv, mask=(h > 0) & keep)
                return bcast_last(ci)

            lax.fori_loop(0, _NB // 16, p2_scatter, zeros, unroll=8)

            def p2_cummax(i, carry):
                v = rlut[pl.ds(i * 16, 16)]
                cm = jnp.maximum(plsc.cummax(v), carry)
                rlut[pl.ds(i * 16, 16)] = cm
                return bcast_last(cm)

            lax.fori_loop(0, _Q // 16, p2_cummax, zeros + tag, unroll=8)

            bbase = b * _NB

            def p2_rmid(i, carry):
                h = hist[pl.ds(bbase + i * 16, 16)]
                ci = plsc.cumsum(h) + carry
                rmid = ci - h + (h >> 1)
                rqf[pl.ds(bbase + i * 16, 16)] = plsc.bitcast(
                    rmid, jnp.float32)
                return bcast_last(ci)

            lax.fori_loop(0, _NB // 16, p2_rmid, zeros, unroll=8)

            @plsc.parallel_loop(0, _NB // 16, unroll=8)
            def _(i):
                rmid = plsc.bitcast(
                    rqf[pl.ds(bbase + i * 16, 16)], jnp.int32)
                j = plsc.load_gather(rlut, [rmid >> 2]) - tag
                rqf[pl.ds(bbase + i * 16, 16)] = (
                    _LO + _WIDTH * (j.astype(jnp.float32) + 0.5))

            return 0

        lax.fori_loop(0, _B, p2_row, 0)

        def mix_chunk(ib, ob, bbase, oml):
            @plsc.parallel_loop(0, _CHR)
            def _(r):
                for cc in range(_CVEC):
                    v = ib[r, pl.ds(cc * 16, 16)]
                    tt = jnp.clip((v - _LO) * _INVW, 0.0, float(_NB - 1))
                    k = tt.astype(jnp.int32) + bbase
                    m = plsc.load_gather(rqf, [k])
                    ob[r, pl.ds(cc * 16, 16)] = v + (m * oml - v * oml)

        def p3_row(b, _):
            bbase = b * _NB
            pltpu.sync_copy(lam_hbm.at[b], lamb)
            oml = 1.0 - lamb[pl.ds(0, 16)]
            start_in(b, chan, 0, i0, si0)
            start_in(b, chan, 1, i1, si1)

            def p3_pair(p, _):
                wait_in(b, chan, 2 * p, i0, si0)

                @pl.when(p > 0)
                def _():
                    wait_out(b, chan, 2 * p - 2, o0, so0)

                mix_chunk(i0, o0, bbase, oml)
                start_out(b, chan, 2 * p, o0, so0)

                @pl.when(2 * p + 2 < _NCH)
                def _():
                    start_in(b, chan, 2 * p + 2, i0, si0)

                wait_in(b, chan, 2 * p + 1, i1, si1)

                @pl.when(p > 0)
                def _():
                    wait_out(b, chan, 2 * p - 1, o1, so1)

                mix_chunk(i1, o1, bbase, oml)
                start_out(b, chan, 2 * p + 1, o1, so1)

                @pl.when(2 * p + 3 < _NCH)
                def _():
                    start_in(b, chan, 2 * p + 3, i1, si1)

                return 0

            lax.fori_loop(0, _NCH // 2, p3_pair, 0)
            wait_out(b, chan, _NCH - 2, o0, so0)
            wait_out(b, chan, _NCH - 1, o1, so1)
            return 0

        lax.fori_loop(0, _B, p3_row, 0)
        return 0

    lax.fori_loop(0, _CPW, channel_body, 0)


_efd_call = pl.kernel(
    _efd_body,
    out_type=jax.ShapeDtypeStruct((_B, _C, _W, _H), jnp.float32),
    mesh=plsc.VectorSubcoreMesh(core_axis_name="c", subcore_axis_name="s"),
    compiler_params=pltpu.CompilerParams(
        needs_layout_passes=False, use_tc_tiling_on_sc=True),
    scratch_types=[
        pltpu.VMEM((_B * _NB,), jnp.int32),
        pltpu.VMEM((_B * _NB,), jnp.float32),
        pltpu.VMEM((_Q,), jnp.int32),
        pltpu.VMEM((_CHR, _H), jnp.float32),
        pltpu.VMEM((_CHR, _H), jnp.float32),
        pltpu.VMEM((_CHR, _H), jnp.float32),
        pltpu.VMEM((_CHR, _H), jnp.float32),
        pltpu.VMEM((128,), jnp.float32),
        pltpu.VMEM((128,), jnp.int32),
        pltpu.SemaphoreType.DMA,
        pltpu.SemaphoreType.DMA,
        pltpu.SemaphoreType.DMA,
        pltpu.SemaphoreType.DMA,
    ],
)


def kernel(x):
    B, C, W, H = x.shape
    k_beta, k_perm = jax.random.split(jax.random.key(42))
    lmda = jax.random.beta(k_beta, 0.1, 0.1, (B, 1, 1)).astype(x.dtype)
    perm = jax.random.permutation(k_perm, B)
    lam128 = jnp.broadcast_to(
        lmda.reshape(B, 1), (B, 128)).astype(jnp.float32)
    perm128 = jnp.broadcast_to(
        perm.reshape(B, 1).astype(jnp.int32), (B, 128))
    return _efd_call(x, lam128, perm128)
